# baseline (device time: 198920 ns/iter reference)
import jax
import jax.numpy as jnp
from jax import lax
from jax.experimental import pallas as pl
from jax.experimental.pallas import tpu as pltpu

N_DEV = 4
N_HOP = N_DEV - 1
FROM_LEFT, FROM_RIGHT, FROM_DIAG = 0, 1, 2


def kernel(x, w_mat):
    m, k = x.shape
    _, n_per = w_mat.shape
    kq = k // 4

    def body(x_hbm, w_ref, out_ref, xQ, lane_r, lane_l,
             blk_l, blk_r, blk_d, copy_sems,
             wsend_r, wrecv_r, wsend_l, wrecv_l, a2a_send, a2a_recv):
        p = lax.axis_index("i")
        left = lax.rem(p + N_DEV - 1, N_DEV)
        right = lax.rem(p + 1, N_DEV)
        diag = lax.rem(p + 2, N_DEV)

        copies = []
        for qi in range(4):
            cp = pltpu.make_async_copy(
                x_hbm.at[:, pl.ds(qi * kq, kq)], xQ.at[qi], copy_sems.at[qi]
            )
            cp.start()
            copies.append(cp)

        barrier_sem = pltpu.get_barrier_semaphore()
        for nbr in (left, right):
            pl.semaphore_signal(
                barrier_sem, inc=1,
                device_id=(nbr,), device_id_type=pl.DeviceIdType.MESH,
            )
        pl.semaphore_wait(barrier_sem, 2)

        def w_hop(h, pc, src_r, src_l, dst_slot):
            r = pltpu.make_async_remote_copy(
                src_ref=src_r, dst_ref=lane_r.at[dst_slot, pc],
                send_sem=wsend_r.at[h, pc], recv_sem=wrecv_r.at[h, pc],
                device_id=(right,), device_id_type=pl.DeviceIdType.MESH,
            )
            l = pltpu.make_async_remote_copy(
                src_ref=src_l, dst_ref=lane_l.at[dst_slot, pc],
                send_sem=wsend_l.at[h, pc], recv_sem=wrecv_l.at[h, pc],
                device_id=(left,), device_id_type=pl.DeviceIdType.MESH,
            )
            r.start()
            l.start()
            return r, l

        r00, l00 = w_hop(0, 0, w_ref.at[pl.ds(0, kq)],
                         w_ref.at[pl.ds(2 * kq, kq)], 0)
        r01, l01 = w_hop(0, 1, w_ref.at[pl.ds(kq, kq)],
                         w_ref.at[pl.ds(3 * kq, kq)], 0)

        r00.wait()
        l00.wait()
        r10, l10 = w_hop(1, 0, lane_r.at[0, 0], lane_l.at[0, 0], 1)
        copies[0].wait()
        copies[2].wait()
        blk_l[...] = jnp.dot(xQ[0], lane_r[0, 0],
                             preferred_element_type=jnp.float32)
        blk_r[...] = jnp.dot(xQ[2], lane_l[0, 0],
                             preferred_element_type=jnp.float32)

        r01.wait()
        l01.wait()
        r11, l11 = w_hop(1, 1, lane_r.at[0, 1], lane_l.at[0, 1], 1)
        copies[1].wait()
        copies[3].wait()
        blk_l[...] = blk_l[...] + jnp.dot(xQ[1], lane_r[0, 1],
                                          preferred_element_type=jnp.float32)
        blk_r[...] = blk_r[...] + jnp.dot(xQ[3], lane_l[0, 1],
                                          preferred_element_type=jnp.float32)

        r10.wait()
        l10.wait()
        r20, l20 = w_hop(2, 0, lane_r.at[1, 0], lane_l.at[1, 0], 0)
        blk_d[...] = (
            jnp.dot(xQ[0], lane_r[1, 0], preferred_element_type=jnp.float32)
            + jnp.dot(xQ[2], lane_l[1, 0], preferred_element_type=jnp.float32)
        )

        r11.wait()
        l11.wait()
        r21, l21 = w_hop(2, 1, lane_r.at[1, 1], lane_l.at[1, 1], 0)
        blk_d[...] = jnp.maximum(
            blk_d[...]
            + jnp.dot(xQ[1], lane_r[1, 1], preferred_element_type=jnp.float32)
            + jnp.dot(xQ[3], lane_l[1, 1], preferred_element_type=jnp.float32),
            0.0,
        )
        send_d = pltpu.make_async_remote_copy(
            src_ref=blk_d, dst_ref=out_ref.at[pl.ds(p * m, m)],
            send_sem=a2a_send.at[FROM_DIAG], recv_sem=a2a_recv.at[FROM_DIAG],
            device_id=(diag,), device_id_type=pl.DeviceIdType.MESH,
        )
        send_d.start()

        r20.wait()
        l20.wait()
        blk_r[...] = blk_r[...] + jnp.dot(xQ[0], lane_r[0, 0],
                                          preferred_element_type=jnp.float32)
        blk_l[...] = blk_l[...] + jnp.dot(xQ[2], lane_l[0, 0],
                                          preferred_element_type=jnp.float32)

        r21.wait()
        l21.wait()
        blk_r[...] = jnp.maximum(
            blk_r[...] + jnp.dot(xQ[1], lane_r[0, 1],
                                 preferred_element_type=jnp.float32),
            0.0,
        )
        send_r = pltpu.make_async_remote_copy(
            src_ref=blk_r, dst_ref=out_ref.at[pl.ds(p * m, m)],
            send_sem=a2a_send.at[FROM_LEFT], recv_sem=a2a_recv.at[FROM_LEFT],
            device_id=(right,), device_id_type=pl.DeviceIdType.MESH,
        )
        send_r.start()
        blk_l[...] = jnp.maximum(
            blk_l[...] + jnp.dot(xQ[3], lane_l[0, 1],
                                 preferred_element_type=jnp.float32),
            0.0,
        )
        send_l = pltpu.make_async_remote_copy(
            src_ref=blk_l, dst_ref=out_ref.at[pl.ds(p * m, m)],
            send_sem=a2a_send.at[FROM_RIGHT], recv_sem=a2a_recv.at[FROM_RIGHT],
            device_id=(left,), device_id_type=pl.DeviceIdType.MESH,
        )
        send_l.start()

        own = jnp.dot(xQ[0], w_ref[pl.ds(0, kq), :],
                      preferred_element_type=jnp.float32)
        for qi in range(1, 4):
            own = own + jnp.dot(xQ[qi], w_ref[pl.ds(qi * kq, kq), :],
                                preferred_element_type=jnp.float32)
        out_ref[pl.ds(p * m, m), :] = jnp.maximum(own, 0.0)

        recv_left = pltpu.make_async_remote_copy(
            src_ref=blk_r, dst_ref=out_ref.at[pl.ds(left * m, m)],
            send_sem=a2a_send.at[FROM_LEFT], recv_sem=a2a_recv.at[FROM_LEFT],
            device_id=(right,), device_id_type=pl.DeviceIdType.MESH,
        )
        recv_right = pltpu.make_async_remote_copy(
            src_ref=blk_l, dst_ref=out_ref.at[pl.ds(right * m, m)],
            send_sem=a2a_send.at[FROM_RIGHT], recv_sem=a2a_recv.at[FROM_RIGHT],
            device_id=(left,), device_id_type=pl.DeviceIdType.MESH,
        )
        recv_diag = pltpu.make_async_remote_copy(
            src_ref=blk_d, dst_ref=out_ref.at[pl.ds(diag * m, m)],
            send_sem=a2a_send.at[FROM_DIAG], recv_sem=a2a_recv.at[FROM_DIAG],
            device_id=(diag,), device_id_type=pl.DeviceIdType.MESH,
        )
        send_d.wait_send()
        send_r.wait_send()
        send_l.wait_send()
        recv_left.wait_recv()
        recv_right.wait_recv()
        recv_diag.wait_recv()

    return pl.pallas_call(
        body,
        out_shape=jax.ShapeDtypeStruct((N_DEV * m, n_per), jnp.float32),
        in_specs=[
            pl.BlockSpec(memory_space=pl.ANY),
            pl.BlockSpec(memory_space=pltpu.VMEM),
        ],
        out_specs=pl.BlockSpec(memory_space=pltpu.VMEM),
        scratch_shapes=[
            pltpu.VMEM((4, m, kq), jnp.float32),
            pltpu.VMEM((2, 2, kq, n_per), jnp.float32),
            pltpu.VMEM((2, 2, kq, n_per), jnp.float32),
            pltpu.VMEM((m, n_per), jnp.float32),
            pltpu.VMEM((m, n_per), jnp.float32),
            pltpu.VMEM((m, n_per), jnp.float32),
            pltpu.SemaphoreType.DMA((4,)),
            pltpu.SemaphoreType.DMA((N_HOP, 2)),
            pltpu.SemaphoreType.DMA((N_HOP, 2)),
            pltpu.SemaphoreType.DMA((N_HOP, 2)),
            pltpu.SemaphoreType.DMA((N_HOP, 2)),
            pltpu.SemaphoreType.DMA((3,)),
            pltpu.SemaphoreType.DMA((3,)),
        ],
        compiler_params=pltpu.CompilerParams(
            collective_id=0,
            vmem_limit_bytes=60 * 1024 * 1024,
        ),
    )(x, w_mat)


# device time: 131506 ns/iter; 1.5126x vs baseline; 1.5126x over previous
import jax
import jax.numpy as jnp
from jax import lax
from jax.experimental import pallas as pl
from jax.experimental.pallas import tpu as pltpu

N_DEV = 4
N_HOP = N_DEV - 1
FROM_LEFT, FROM_RIGHT, FROM_DIAG = 0, 1, 2


def kernel(x, w_mat):
    m, k = x.shape
    _, n_per = w_mat.shape
    kq = k // 4

    def body(x_hbm, w_ref, out_ref, xstage, xQ, wQ, lane_r, lane_l,
             blk_l, blk_r, blk_d, copy_sems,
             wsend_r, wrecv_r, wsend_l, wrecv_l, a2a_send, a2a_recv):
        p = lax.axis_index("i")
        left = lax.rem(p + N_DEV - 1, N_DEV)
        right = lax.rem(p + 1, N_DEV)
        diag = lax.rem(p + 2, N_DEV)

        def x_load(qi, slot):
            cp = pltpu.make_async_copy(
                x_hbm.at[:, pl.ds(qi * kq, kq)], xstage.at[slot],
                copy_sems.at[qi],
            )
            cp.start()
            return cp

        cp0 = x_load(0, 0)
        cp2 = x_load(2, 1)

        barrier_sem = pltpu.get_barrier_semaphore()
        for nbr in (left, right):
            pl.semaphore_signal(
                barrier_sem, inc=1,
                device_id=(nbr,), device_id_type=pl.DeviceIdType.MESH,
            )
        pl.semaphore_wait(barrier_sem, 2)

        def w_hop(h, pc, src_r, src_l, dst_slot):
            r = pltpu.make_async_remote_copy(
                src_ref=src_r, dst_ref=lane_r.at[dst_slot, pc],
                send_sem=wsend_r.at[h, pc], recv_sem=wrecv_r.at[h, pc],
                device_id=(right,), device_id_type=pl.DeviceIdType.MESH,
            )
            l = pltpu.make_async_remote_copy(
                src_ref=src_l, dst_ref=lane_l.at[dst_slot, pc],
                send_sem=wsend_l.at[h, pc], recv_sem=wrecv_l.at[h, pc],
                device_id=(left,), device_id_type=pl.DeviceIdType.MESH,
            )
            r.start()
            l.start()
            return r, l

        wQ[0] = w_ref[pl.ds(0, kq), :].astype(jnp.bfloat16)
        wQ[2] = w_ref[pl.ds(2 * kq, kq), :].astype(jnp.bfloat16)
        r00, l00 = w_hop(0, 0, wQ.at[0], wQ.at[2], 0)
        wQ[1] = w_ref[pl.ds(kq, kq), :].astype(jnp.bfloat16)
        wQ[3] = w_ref[pl.ds(3 * kq, kq), :].astype(jnp.bfloat16)
        r01, l01 = w_hop(0, 1, wQ.at[1], wQ.at[3], 0)

        cp0.wait()
        xQ[0] = xstage[0].astype(jnp.bfloat16)
        cp1 = x_load(1, 0)
        cp2.wait()
        xQ[2] = xstage[1].astype(jnp.bfloat16)
        cp3 = x_load(3, 1)
        cp1.wait()
        xQ[1] = xstage[0].astype(jnp.bfloat16)
        cp3.wait()
        xQ[3] = xstage[1].astype(jnp.bfloat16)

        r00.wait()
        l00.wait()
        r10, l10 = w_hop(1, 0, lane_r.at[0, 0], lane_l.at[0, 0], 1)
        blk_l[...] = jnp.dot(xQ[0], lane_r[0, 0],
                             preferred_element_type=jnp.float32)
        blk_r[...] = jnp.dot(xQ[2], lane_l[0, 0],
                             preferred_element_type=jnp.float32)

        r01.wait()
        l01.wait()
        r11, l11 = w_hop(1, 1, lane_r.at[0, 1], lane_l.at[0, 1], 1)
        blk_l[...] = blk_l[...] + jnp.dot(xQ[1], lane_r[0, 1],
                                          preferred_element_type=jnp.float32)
        blk_r[...] = blk_r[...] + jnp.dot(xQ[3], lane_l[0, 1],
                                          preferred_element_type=jnp.float32)

        r10.wait()
        l10.wait()
        r20, l20 = w_hop(2, 0, lane_r.at[1, 0], lane_l.at[1, 0], 0)
        blk_d[...] = (
            jnp.dot(xQ[0], lane_r[1, 0], preferred_element_type=jnp.float32)
            + jnp.dot(xQ[2], lane_l[1, 0], preferred_element_type=jnp.float32)
        )

        r11.wait()
        l11.wait()
        r21, l21 = w_hop(2, 1, lane_r.at[1, 1], lane_l.at[1, 1], 0)
        blk_d[...] = jnp.maximum(
            blk_d[...]
            + jnp.dot(xQ[1], lane_r[1, 1], preferred_element_type=jnp.float32)
            + jnp.dot(xQ[3], lane_l[1, 1], preferred_element_type=jnp.float32),
            0.0,
        )
        send_d = pltpu.make_async_remote_copy(
            src_ref=blk_d, dst_ref=out_ref.at[pl.ds(p * m, m)],
            send_sem=a2a_send.at[FROM_DIAG], recv_sem=a2a_recv.at[FROM_DIAG],
            device_id=(diag,), device_id_type=pl.DeviceIdType.MESH,
        )
        send_d.start()

        r20.wait()
        l20.wait()
        blk_r[...] = blk_r[...] + jnp.dot(xQ[0], lane_r[0, 0],
                                          preferred_element_type=jnp.float32)
        blk_l[...] = blk_l[...] + jnp.dot(xQ[2], lane_l[0, 0],
                                          preferred_element_type=jnp.float32)

        r21.wait()
        l21.wait()
        blk_r[...] = jnp.maximum(
            blk_r[...] + jnp.dot(xQ[1], lane_r[0, 1],
                                 preferred_element_type=jnp.float32),
            0.0,
        )
        send_r = pltpu.make_async_remote_copy(
            src_ref=blk_r, dst_ref=out_ref.at[pl.ds(p * m, m)],
            send_sem=a2a_send.at[FROM_LEFT], recv_sem=a2a_recv.at[FROM_LEFT],
            device_id=(right,), device_id_type=pl.DeviceIdType.MESH,
        )
        send_r.start()
        blk_l[...] = jnp.maximum(
            blk_l[...] + jnp.dot(xQ[3], lane_l[0, 1],
                                 preferred_element_type=jnp.float32),
            0.0,
        )
        send_l = pltpu.make_async_remote_copy(
            src_ref=blk_l, dst_ref=out_ref.at[pl.ds(p * m, m)],
            send_sem=a2a_send.at[FROM_RIGHT], recv_sem=a2a_recv.at[FROM_RIGHT],
            device_id=(left,), device_id_type=pl.DeviceIdType.MESH,
        )
        send_l.start()

        own = jnp.dot(xQ[0], wQ[0], preferred_element_type=jnp.float32)
        for qi in range(1, 4):
            own = own + jnp.dot(xQ[qi], wQ[qi],
                                preferred_element_type=jnp.float32)
        out_ref[pl.ds(p * m, m), :] = jnp.maximum(own, 0.0)

        recv_left = pltpu.make_async_remote_copy(
            src_ref=blk_r, dst_ref=out_ref.at[pl.ds(left * m, m)],
            send_sem=a2a_send.at[FROM_LEFT], recv_sem=a2a_recv.at[FROM_LEFT],
            device_id=(right,), device_id_type=pl.DeviceIdType.MESH,
        )
        recv_right = pltpu.make_async_remote_copy(
            src_ref=blk_l, dst_ref=out_ref.at[pl.ds(right * m, m)],
            send_sem=a2a_send.at[FROM_RIGHT], recv_sem=a2a_recv.at[FROM_RIGHT],
            device_id=(left,), device_id_type=pl.DeviceIdType.MESH,
        )
        recv_diag = pltpu.make_async_remote_copy(
            src_ref=blk_d, dst_ref=out_ref.at[pl.ds(diag * m, m)],
            send_sem=a2a_send.at[FROM_DIAG], recv_sem=a2a_recv.at[FROM_DIAG],
            device_id=(diag,), device_id_type=pl.DeviceIdType.MESH,
        )
        send_d.wait_send()
        send_r.wait_send()
        send_l.wait_send()
        recv_left.wait_recv()
        recv_right.wait_recv()
        recv_diag.wait_recv()

    return pl.pallas_call(
        body,
        out_shape=jax.ShapeDtypeStruct((N_DEV * m, n_per), jnp.float32),
        in_specs=[
            pl.BlockSpec(memory_space=pl.ANY),
            pl.BlockSpec(memory_space=pltpu.VMEM),
        ],
        out_specs=pl.BlockSpec(memory_space=pltpu.VMEM),
        scratch_shapes=[
            pltpu.VMEM((2, m, kq), jnp.float32),
            pltpu.VMEM((4, m, kq), jnp.bfloat16),
            pltpu.VMEM((4, kq, n_per), jnp.bfloat16),
            pltpu.VMEM((2, 2, kq, n_per), jnp.bfloat16),
            pltpu.VMEM((2, 2, kq, n_per), jnp.bfloat16),
            pltpu.VMEM((m, n_per), jnp.float32),
            pltpu.VMEM((m, n_per), jnp.float32),
            pltpu.VMEM((m, n_per), jnp.float32),
            pltpu.SemaphoreType.DMA((4,)),
            pltpu.SemaphoreType.DMA((N_HOP, 2)),
            pltpu.SemaphoreType.DMA((N_HOP, 2)),
            pltpu.SemaphoreType.DMA((N_HOP, 2)),
            pltpu.SemaphoreType.DMA((N_HOP, 2)),
            pltpu.SemaphoreType.DMA((3,)),
            pltpu.SemaphoreType.DMA((3,)),
        ],
        compiler_params=pltpu.CompilerParams(
            collective_id=0,
            vmem_limit_bytes=60 * 1024 * 1024,
        ),
    )(x, w_mat)


# device time: 109292 ns/iter; 1.8201x vs baseline; 1.2033x over previous
import jax
import jax.numpy as jnp
from jax import lax
from jax.experimental import pallas as pl
from jax.experimental.pallas import tpu as pltpu

N_DEV = 4
N_HOP = N_DEV - 1
FROM_LEFT, FROM_RIGHT, FROM_DIAG = 0, 1, 2


def kernel(x, w_mat):
    m, k = x.shape
    _, n_per = w_mat.shape
    kq = k // 4

    def body(x_hbm, w_ref, out_ref, xstage, xQ, wQ, lane_r, lane_l,
             blk_l, blk_r, blk_d, sb, rb, copy_sems,
             wsend_r, wrecv_r, wsend_l, wrecv_l, a2a_send, a2a_recv):
        p = lax.axis_index("i")
        left = lax.rem(p + N_DEV - 1, N_DEV)
        right = lax.rem(p + 1, N_DEV)
        diag = lax.rem(p + 2, N_DEV)

        def x_load(qi):
            cp = pltpu.make_async_copy(
                x_hbm.at[:, pl.ds(qi * kq, kq)], xstage, copy_sems.at[qi]
            )
            cp.start()
            return cp

        cp0 = x_load(0)

        barrier_sem = pltpu.get_barrier_semaphore()
        for nbr in (left, right):
            pl.semaphore_signal(
                barrier_sem, inc=1,
                device_id=(nbr,), device_id_type=pl.DeviceIdType.MESH,
            )
        pl.semaphore_wait(barrier_sem, 2)

        def w_hop(h, pc, src_r, src_l, dst_slot):
            r = pltpu.make_async_remote_copy(
                src_ref=src_r, dst_ref=lane_r.at[dst_slot, pc],
                send_sem=wsend_r.at[h, pc], recv_sem=wrecv_r.at[h, pc],
                device_id=(right,), device_id_type=pl.DeviceIdType.MESH,
            )
            l = pltpu.make_async_remote_copy(
                src_ref=src_l, dst_ref=lane_l.at[dst_slot, pc],
                send_sem=wsend_l.at[h, pc], recv_sem=wrecv_l.at[h, pc],
                device_id=(left,), device_id_type=pl.DeviceIdType.MESH,
            )
            r.start()
            l.start()
            return r, l

        wQ[0] = w_ref[pl.ds(0, kq), :].astype(jnp.bfloat16)
        wQ[2] = w_ref[pl.ds(2 * kq, kq), :].astype(jnp.bfloat16)
        r00, l00 = w_hop(0, 0, wQ.at[0], wQ.at[2], 0)
        wQ[1] = w_ref[pl.ds(kq, kq), :].astype(jnp.bfloat16)
        wQ[3] = w_ref[pl.ds(3 * kq, kq), :].astype(jnp.bfloat16)
        r01, l01 = w_hop(0, 1, wQ.at[1], wQ.at[3], 0)

        cp0.wait()
        xQ[0] = xstage[...].astype(jnp.bfloat16)
        cp2 = x_load(2)
        cp2.wait()
        xQ[2] = xstage[...].astype(jnp.bfloat16)
        cp1 = x_load(1)

        r00.wait()
        l00.wait()
        r10, l10 = w_hop(1, 0, lane_r.at[0, 0], lane_l.at[0, 0], 1)
        blk_l[...] = jnp.dot(xQ[0], lane_r[0, 0],
                             preferred_element_type=jnp.float32)
        blk_r[...] = jnp.dot(xQ[2], lane_l[0, 0],
                             preferred_element_type=jnp.float32)
        cp1.wait()
        xQ[1] = xstage[...].astype(jnp.bfloat16)
        cp3 = x_load(3)
        cp3.wait()
        xQ[3] = xstage[...].astype(jnp.bfloat16)

        r01.wait()
        l01.wait()
        r11, l11 = w_hop(1, 1, lane_r.at[0, 1], lane_l.at[0, 1], 1)
        blk_l[...] = blk_l[...] + jnp.dot(xQ[1], lane_r[0, 1],
                                          preferred_element_type=jnp.float32)
        blk_r[...] = blk_r[...] + jnp.dot(xQ[3], lane_l[0, 1],
                                          preferred_element_type=jnp.float32)

        r10.wait()
        l10.wait()
        r20, l20 = w_hop(2, 0, lane_r.at[1, 0], lane_l.at[1, 0], 0)
        blk_d[...] = (
            jnp.dot(xQ[0], lane_r[1, 0], preferred_element_type=jnp.float32)
            + jnp.dot(xQ[2], lane_l[1, 0], preferred_element_type=jnp.float32)
        )

        r11.wait()
        l11.wait()
        r21, l21 = w_hop(2, 1, lane_r.at[1, 1], lane_l.at[1, 1], 0)
        sb[FROM_DIAG] = jnp.maximum(
            blk_d[...]
            + jnp.dot(xQ[1], lane_r[1, 1], preferred_element_type=jnp.float32)
            + jnp.dot(xQ[3], lane_l[1, 1], preferred_element_type=jnp.float32),
            0.0,
        ).astype(jnp.bfloat16)
        send_d = pltpu.make_async_remote_copy(
            src_ref=sb.at[FROM_DIAG], dst_ref=rb.at[FROM_DIAG],
            send_sem=a2a_send.at[FROM_DIAG], recv_sem=a2a_recv.at[FROM_DIAG],
            device_id=(diag,), device_id_type=pl.DeviceIdType.MESH,
        )
        send_d.start()

        r20.wait()
        l20.wait()
        blk_r[...] = blk_r[...] + jnp.dot(xQ[0], lane_r[0, 0],
                                          preferred_element_type=jnp.float32)
        blk_l[...] = blk_l[...] + jnp.dot(xQ[2], lane_l[0, 0],
                                          preferred_element_type=jnp.float32)

        r21.wait()
        l21.wait()
        sb[FROM_LEFT] = jnp.maximum(
            blk_r[...] + jnp.dot(xQ[1], lane_r[0, 1],
                                 preferred_element_type=jnp.float32),
            0.0,
        ).astype(jnp.bfloat16)
        send_r = pltpu.make_async_remote_copy(
            src_ref=sb.at[FROM_LEFT], dst_ref=rb.at[FROM_LEFT],
            send_sem=a2a_send.at[FROM_LEFT], recv_sem=a2a_recv.at[FROM_LEFT],
            device_id=(right,), device_id_type=pl.DeviceIdType.MESH,
        )
        send_r.start()
        sb[FROM_RIGHT] = jnp.maximum(
            blk_l[...] + jnp.dot(xQ[3], lane_l[0, 1],
                                 preferred_element_type=jnp.float32),
            0.0,
        ).astype(jnp.bfloat16)
        send_l = pltpu.make_async_remote_copy(
            src_ref=sb.at[FROM_RIGHT], dst_ref=rb.at[FROM_RIGHT],
            send_sem=a2a_send.at[FROM_RIGHT], recv_sem=a2a_recv.at[FROM_RIGHT],
            device_id=(left,), device_id_type=pl.DeviceIdType.MESH,
        )
        send_l.start()

        own = jnp.dot(xQ[0], wQ[0], preferred_element_type=jnp.float32)
        for qi in range(1, 4):
            own = own + jnp.dot(xQ[qi], wQ[qi],
                                preferred_element_type=jnp.float32)
        out_ref[pl.ds(p * m, m), :] = jnp.maximum(own, 0.0)

        def a2a_recv_desc(slot):
            return pltpu.make_async_remote_copy(
                src_ref=sb.at[slot], dst_ref=rb.at[slot],
                send_sem=a2a_send.at[slot], recv_sem=a2a_recv.at[slot],
                device_id=(diag,), device_id_type=pl.DeviceIdType.MESH,
            )

        send_d.wait_send()
        send_r.wait_send()
        send_l.wait_send()
        a2a_recv_desc(FROM_LEFT).wait_recv()
        out_ref[pl.ds(left * m, m), :] = rb[FROM_LEFT].astype(jnp.float32)
        a2a_recv_desc(FROM_RIGHT).wait_recv()
        out_ref[pl.ds(right * m, m), :] = rb[FROM_RIGHT].astype(jnp.float32)
        a2a_recv_desc(FROM_DIAG).wait_recv()
        out_ref[pl.ds(diag * m, m), :] = rb[FROM_DIAG].astype(jnp.float32)

    return pl.pallas_call(
        body,
        out_shape=jax.ShapeDtypeStruct((N_DEV * m, n_per), jnp.float32),
        in_specs=[
            pl.BlockSpec(memory_space=pl.ANY),
            pl.BlockSpec(memory_space=pltpu.VMEM),
        ],
        out_specs=pl.BlockSpec(memory_space=pltpu.VMEM),
        scratch_shapes=[
            pltpu.VMEM((m, kq), jnp.float32),
            pltpu.VMEM((4, m, kq), jnp.bfloat16),
            pltpu.VMEM((4, kq, n_per), jnp.bfloat16),
            pltpu.VMEM((2, 2, kq, n_per), jnp.bfloat16),
            pltpu.VMEM((2, 2, kq, n_per), jnp.bfloat16),
            pltpu.VMEM((m, n_per), jnp.float32),
            pltpu.VMEM((m, n_per), jnp.float32),
            pltpu.VMEM((m, n_per), jnp.float32),
            pltpu.VMEM((3, m, n_per), jnp.bfloat16),
            pltpu.VMEM((3, m, n_per), jnp.bfloat16),
            pltpu.SemaphoreType.DMA((4,)),
            pltpu.SemaphoreType.DMA((N_HOP, 2)),
            pltpu.SemaphoreType.DMA((N_HOP, 2)),
            pltpu.SemaphoreType.DMA((N_HOP, 2)),
            pltpu.SemaphoreType.DMA((N_HOP, 2)),
            pltpu.SemaphoreType.DMA((3,)),
            pltpu.SemaphoreType.DMA((3,)),
        ],
        compiler_params=pltpu.CompilerParams(
            collective_id=0,
            vmem_limit_bytes=60 * 1024 * 1024,
        ),
    )(x, w_mat)
